# baseline (speedup 1.0000x reference)
import jax
import jax.numpy as jnp
from jax.experimental import pallas as pl

_TOP_K = 4
_EPS = 1e-06
_SPLITC = 4097.0


def _split3(x):
    hi = x.astype(jnp.bfloat16)
    lo = (x - hi.astype(jnp.float32)).astype(jnp.bfloat16)
    return hi, lo


def _dekker_split(x):
    c = x * _SPLITC
    h = c - (c - x)
    return h, x - h


def _exact_dot_rows(a, b):
    ah, al = _dekker_split(a)
    bh, bl = _dekker_split(b)
    s = a * b
    e = ((ah * bh - s) + ah * bl + al * bh) + al * bl
    w = a.shape[1]
    while w > 1:
        h = w // 2
        x = s[:, :h]
        y = s[:, h:w]
        t = x + y
        yy = t - x
        err = (x - (t - yy)) + (y - yy)
        e = e[:, :h] + e[:, h:w] + err
        s = t
        w = h
    return s + e


def _obm_kernel(k_ref, v_ref, q_ref, o_ref):
    k = k_ref[0, 0]
    v = v_ref[0, 0]
    q = q_ref[0, 0]
    S, D = k.shape
    iota = jax.lax.broadcasted_iota(jnp.int32, (S, D), 1)

    ak = jnp.abs(k)
    kmax = jnp.max(ak, axis=1, keepdims=True)
    a_idx = jnp.min(jnp.where(ak == kmax, iota, D), axis=1, keepdims=True)
    pt = (iota == a_idx).astype(jnp.float32)

    hi = jax.lax.Precision.HIGHEST
    dn = (((1,), (1,)), ((), ()))
    dc = (((1,), (0,)), ((), ()))

    z = jax.lax.dot_general(pt, k, (((0,), (0,)), ((), ())),
                            preferred_element_type=jnp.float32, precision=hi)

    qh, ql = _split3(q)
    kh, kl = _split3(k)
    qcat = jnp.concatenate([qh, qh, ql], axis=1)
    kcat = jnp.concatenate([kh, kl, kh], axis=1)
    A = jax.lax.dot_general(qcat, kcat, dn, preferred_element_type=jnp.float32)

    work = jnp.abs(q)
    scores = []
    onehots = []
    for _ in range(_TOP_K):
        m = jnp.max(work, axis=1, keepdims=True)
        idx = jnp.min(jnp.where(work == m, iota, D), axis=1, keepdims=True)
        oh = iota == idx
        scores.append(m)
        onehots.append(oh)
        work = jnp.where(oh, -jnp.inf, work)
    es = [jnp.exp(s - scores[0]) for s in scores]
    w_norm = es[0]
    for e in es[1:]:
        w_norm = w_norm + e

    g = jnp.zeros((S, D), jnp.float32)
    for e, oh in zip(es, onehots):
        zsel = jax.lax.dot_general(oh.astype(jnp.float32), z, dc,
                                   preferred_element_type=jnp.float32,
                                   precision=hi)
        seg = _exact_dot_rows(zsel, q) + _EPS
        g = g + jnp.where(oh, (e / w_norm) / seg, 0.0)

    gh, gl = _split3(g)
    ptb = pt.astype(jnp.bfloat16)
    gcat = jnp.concatenate([gh, gl], axis=1)
    ptcat = jnp.concatenate([ptb, ptb], axis=1)
    G = jax.lax.dot_general(gcat, ptcat, dn, preferred_element_type=jnp.float32)

    C = A * G
    ch, cl = _split3(C)
    vh, vl = _split3(v)
    o = jax.lax.dot_general(ch, vh, dc, preferred_element_type=jnp.float32)
    o = o + jax.lax.dot_general(ch, vl, dc, preferred_element_type=jnp.float32)
    o = o + jax.lax.dot_general(cl, vh, dc, preferred_element_type=jnp.float32)
    o_ref[0, 0] = o


@jax.jit
def kernel(keys, values, queries):
    Bb, H, S, D = keys.shape
    spec = pl.BlockSpec((1, 1, S, D), lambda b, h: (b, h, 0, 0))
    return pl.pallas_call(
        _obm_kernel,
        grid=(Bb, H),
        in_specs=[spec, spec, spec],
        out_specs=spec,
        out_shape=jax.ShapeDtypeStruct((Bb, H, S, D), jnp.float32),
    )(keys, values, queries)


# speedup vs baseline: 2.3554x; 2.3554x over previous
import jax
import jax.numpy as jnp
from jax.experimental import pallas as pl

_TOP_K = 4
_EPS = 1e-06


def _split3(x):
    hi = x.astype(jnp.bfloat16)
    lo = (x - hi.astype(jnp.float32)).astype(jnp.bfloat16)
    return hi, lo


def _obm_kernel(k_ref, v_ref, q_ref, o_ref):
    k = k_ref[0, 0]
    v = v_ref[0, 0]
    q = q_ref[0, 0]
    S, D = k.shape
    niota = -jax.lax.broadcasted_iota(jnp.int32, (S, D), 1).astype(jnp.float32)

    ak = jnp.abs(k)
    kmax = jnp.max(ak, axis=1, keepdims=True)
    nsel = jnp.where(ak == kmax, niota, -jnp.inf)
    pt = (nsel == jnp.max(nsel, axis=1, keepdims=True)).astype(jnp.float32)

    hi = jax.lax.Precision.HIGHEST
    dn = (((1,), (1,)), ((), ()))
    dc = (((1,), (0,)), ((), ()))

    z = jax.lax.dot_general(pt, k, (((0,), (0,)), ((), ())),
                            preferred_element_type=jnp.float32, precision=hi)

    qh, ql = _split3(q)
    kh, kl = _split3(k)
    qcat = jnp.concatenate([qh, qh, ql], axis=1)
    kcat = jnp.concatenate([kh, kl, kh], axis=1)
    A = jax.lax.dot_general(qcat, kcat, dn, preferred_element_type=jnp.float32)

    work = jnp.abs(q)
    scores = []
    onehots = []
    for _ in range(_TOP_K):
        m = jnp.max(work, axis=1, keepdims=True)
        ns = jnp.where(work == m, niota, -jnp.inf)
        oh = ns == jnp.max(ns, axis=1, keepdims=True)
        scores.append(m)
        onehots.append(oh)
        work = jnp.where(oh, -jnp.inf, work)
    es = [jnp.exp(s - scores[0]) for s in scores]
    w_norm = es[0]
    for e in es[1:]:
        w_norm = w_norm + e

    asum = jax.lax.dot_general(q, z, dn,
                               preferred_element_type=jnp.float32, precision=hi)

    g = jnp.zeros((S, D), jnp.float32)
    for e, oh in zip(es, onehots):
        seg = jnp.sum(jnp.where(oh, asum, 0.0), axis=1, keepdims=True) + _EPS
        g = g + jnp.where(oh, (e / w_norm) / seg, 0.0)

    gh, gl = _split3(g)
    ptb = pt.astype(jnp.bfloat16)
    gcat = jnp.concatenate([gh, gl], axis=1)
    ptcat = jnp.concatenate([ptb, ptb], axis=1)
    G = jax.lax.dot_general(gcat, ptcat, dn, preferred_element_type=jnp.float32)

    C = A * G
    ch, cl = _split3(C)
    vh, vl = _split3(v)
    o = jax.lax.dot_general(ch, vh, dc, preferred_element_type=jnp.float32)
    o = o + jax.lax.dot_general(ch, vl, dc, preferred_element_type=jnp.float32)
    o = o + jax.lax.dot_general(cl, vh, dc, preferred_element_type=jnp.float32)
    o_ref[0, 0] = o


@jax.jit
def kernel(keys, values, queries):
    Bb, H, S, D = keys.shape
    spec = pl.BlockSpec((1, 1, S, D), lambda b, h: (b, h, 0, 0))
    return pl.pallas_call(
        _obm_kernel,
        grid=(Bb, H),
        in_specs=[spec, spec, spec],
        out_specs=spec,
        out_shape=jax.ShapeDtypeStruct((Bb, H, S, D), jnp.float32),
    )(keys, values, queries)


# speedup vs baseline: 2.6685x; 1.1329x over previous
import jax
import jax.numpy as jnp
from jax.experimental import pallas as pl

_TOP_K = 4
_EPS = 1e-06


def _split3(x):
    hi = x.astype(jnp.bfloat16)
    lo = (x - hi.astype(jnp.float32)).astype(jnp.bfloat16)
    return hi, lo


def _obm_kernel(k_ref, v_ref, q_ref, o_ref):
    k = k_ref[...]
    v = v_ref[...]
    q = q_ref[...]
    S, D = k.shape
    niota = -jax.lax.broadcasted_iota(jnp.int32, (S, D), 1).astype(jnp.float32)

    ak = jnp.abs(k)
    kmax = jnp.max(ak, axis=1, keepdims=True)
    nsel = jnp.where(ak == kmax, niota, -jnp.inf)
    pt = (nsel == jnp.max(nsel, axis=1, keepdims=True)).astype(jnp.float32)

    hi = jax.lax.Precision.HIGHEST
    dn = (((1,), (1,)), ((), ()))
    dc = (((1,), (0,)), ((), ()))

    z = jax.lax.dot_general(pt, k, (((0,), (0,)), ((), ())),
                            preferred_element_type=jnp.float32, precision=hi)

    qh, ql = _split3(q)
    kh, kl = _split3(k)
    qcat = jnp.concatenate([qh, qh, ql], axis=1)
    kcat = jnp.concatenate([kh, kl, kh], axis=1)
    A = jax.lax.dot_general(qcat, kcat, dn, preferred_element_type=jnp.float32)

    work = jnp.abs(q)
    scores = []
    onehots = []
    for _ in range(_TOP_K):
        m = jnp.max(work, axis=1, keepdims=True)
        ns = jnp.where(work == m, niota, -jnp.inf)
        oh = ns == jnp.max(ns, axis=1, keepdims=True)
        scores.append(m)
        onehots.append(oh)
        work = jnp.where(oh, -jnp.inf, work)
    es = [jnp.exp(s - scores[0]) for s in scores]
    w_norm = es[0]
    for e in es[1:]:
        w_norm = w_norm + e

    asum = jax.lax.dot_general(q, z, dn,
                               preferred_element_type=jnp.float32, precision=hi)

    g = jnp.zeros((S, D), jnp.float32)
    for e, oh in zip(es, onehots):
        seg = jnp.sum(jnp.where(oh, asum, 0.0), axis=1, keepdims=True) + _EPS
        g = g + jnp.where(oh, (e / w_norm) / seg, 0.0)

    gh, gl = _split3(g)
    ptb = pt.astype(jnp.bfloat16)
    gcat = jnp.concatenate([gh, gl], axis=1)
    ptcat = jnp.concatenate([ptb, ptb], axis=1)
    G = jax.lax.dot_general(gcat, ptcat, dn, preferred_element_type=jnp.float32)

    C = A * G
    ch, cl = _split3(C)
    vh, vl = _split3(v)
    o = jax.lax.dot_general(ch, vh, dc, preferred_element_type=jnp.float32)
    o = o + jax.lax.dot_general(ch, vl, dc, preferred_element_type=jnp.float32)
    o = o + jax.lax.dot_general(cl, vh, dc, preferred_element_type=jnp.float32)
    o_ref[...] = o


def _obm_all(k_ref, v_ref, q_ref, o_ref):
    Bb, H = k_ref.shape[:2]
    for b in range(Bb):
        for h in range(H):
            _obm_kernel(k_ref.at[b, h], v_ref.at[b, h], q_ref.at[b, h],
                        o_ref.at[b, h])


@jax.jit
def kernel(keys, values, queries):
    Bb, H, S, D = keys.shape
    return pl.pallas_call(
        _obm_all,
        out_shape=jax.ShapeDtypeStruct((Bb, H, S, D), jnp.float32),
    )(keys, values, queries)
